# baseline (speedup 1.0000x reference)
import functools

import jax
import jax.numpy as jnp
from jax import lax
from jax.experimental import pallas as pl
from jax.experimental.pallas import tpu as pltpu
from jax.experimental.pallas import tpu_sc as plsc

N = 10000
D = 128
E = 320000
NC = 2
NS = 16
NW = NC * NS
CH = 128
NCH = 80
EP = NW * NCH * CH
NP = 10240
RPT = NP // NS
PAD_DST = NP - 1
SHIFT = 14


def _sc_scatter_add(h, pk3, zrows):
    mesh = plsc.VectorSubcoreMesh(
        core_axis_name="c", subcore_axis_name="s", num_cores=NC, num_subcores=NS
    )

    @functools.partial(
        pl.kernel,
        out_type=jax.ShapeDtypeStruct((NC, NP, D), jnp.float32),
        mesh=mesh,
        scratch_types=[
            pltpu.VMEM((NCH, CH), jnp.int32),
            pltpu.VMEM((2, CH), jnp.int32),
            pltpu.VMEM((2, CH), jnp.int32),
            pltpu.VMEM((2, CH, D), jnp.float32),
            pltpu.VMEM_SHARED((NP, D), jnp.float32),
            pltpu.SemaphoreType.DMA((2,)),
            pltpu.SemaphoreType.DMA((2,)),
        ],
    )
    def k(h_hbm, pk_hbm, z_hbm, out_hbm,
          pk_v, srcb_v, dstb_v, rows_v, acc_sh, gsem, ssem):
        c = lax.axis_index("c")
        s = lax.axis_index("s")
        wid = c * NS + s
        pltpu.sync_copy(pk_hbm.at[wid], pk_v)
        pltpu.sync_copy(z_hbm.at[pl.ds(s * RPT, RPT)],
                        acc_sh.at[pl.ds(s * RPT, RPT)])
        plsc.subcore_barrier()

        def unpack(j, slot):
            for kk in range(CH // 16):
                v = pk_v[j, pl.ds(kk * 16, 16)]
                srcb_v[slot, pl.ds(kk * 16, 16)] = v & ((1 << SHIFT) - 1)
                dstb_v[slot, pl.ds(kk * 16, 16)] = lax.shift_right_logical(
                    v, SHIFT)

        def gather_start(b):
            pltpu.async_copy(h_hbm.at[srcb_v.at[b]], rows_v.at[b], gsem.at[b])

        def gather_wait(b):
            pltpu.make_async_copy(
                h_hbm.at[srcb_v.at[b]], rows_v.at[b], gsem.at[b]).wait()

        def scatter_start(b):
            pltpu.async_copy(
                rows_v.at[b], acc_sh.at[dstb_v.at[b]], ssem.at[b], add=True)

        def scatter_wait(b):
            pltpu.make_async_copy(
                rows_v.at[b], acc_sh.at[dstb_v.at[b]], ssem.at[b]).wait()

        unpack(0, 0)
        gather_start(0)

        def body(j, carry):
            b = lax.rem(j, 2)
            nb = lax.rem(j + 1, 2)

            @pl.when(j >= 1)
            def _():
                scatter_wait(nb)

            @pl.when(j + 1 < NCH)
            def _():
                unpack(j + 1, nb)
                gather_start(nb)

            gather_wait(b)
            scatter_start(b)
            return carry

        lax.fori_loop(0, NCH, body, 0)
        scatter_wait(lax.rem(NCH - 1, 2))
        plsc.subcore_barrier()
        pltpu.sync_copy(acc_sh.at[pl.ds(s * RPT, RPT)],
                        out_hbm.at[c, pl.ds(s * RPT, RPT)])

    return k(h, pk3, zrows)


BR = 1000


def _tc_specs(n_extra):
    row = pl.BlockSpec((BR, D), lambda i: (i, 0))
    full = pl.BlockSpec((1, D), lambda i: (0, 0))
    return (
        [row, pl.BlockSpec((NC, BR, D), lambda i: (0, i, 0))]
        + [pl.BlockSpec((D, D), lambda i: (0, 0)), full] * (2 + n_extra // 2),
        row,
    )


def _mlp(h, p, W1, b1, W2, b2):
    def body(h_ref, p_ref, w1_ref, b1_ref, w2_ref, b2_ref, o_ref):
        u = h_ref[...] + p_ref[0] + p_ref[1]
        t = jnp.dot(u, w1_ref[...], preferred_element_type=jnp.float32)
        t = jnp.maximum(t + b1_ref[...], 0.0)
        v = jnp.dot(t, w2_ref[...], preferred_element_type=jnp.float32)
        o_ref[...] = jnp.maximum(v + b2_ref[...], 0.0)

    in_specs, out_spec = _tc_specs(0)
    return pl.pallas_call(
        body,
        grid=(N // BR,),
        in_specs=in_specs,
        out_specs=out_spec,
        out_shape=jax.ShapeDtypeStruct((N, D), jnp.float32),
    )(h, p, W1, b1, W2, b2)


def _mlp_fc(h, p, W1, b1, W2, b2, Wfcp, bfcp):
    def body(h_ref, p_ref, w1_ref, b1_ref, w2_ref, b2_ref,
             wfc_ref, bfc_ref, o_ref):
        u = h_ref[...] + p_ref[0] + p_ref[1]
        t = jnp.dot(u, w1_ref[...], preferred_element_type=jnp.float32)
        t = jnp.maximum(t + b1_ref[...], 0.0)
        v = jnp.dot(t, w2_ref[...], preferred_element_type=jnp.float32)
        h2 = jnp.maximum(v + b2_ref[...], 0.0)
        o = jnp.dot(h2, wfc_ref[...], preferred_element_type=jnp.float32)
        o_ref[...] = o + bfc_ref[...]

    in_specs, out_spec = _tc_specs(2)
    return pl.pallas_call(
        body,
        grid=(N // BR,),
        in_specs=in_specs,
        out_specs=out_spec,
        out_shape=jax.ShapeDtypeStruct((N, D), jnp.float32),
    )(h, p, W1, b1, W2, b2, Wfcp, bfcp)


def kernel(x, edge_index, W1a, b1a, W2a, b2a, W1b, b1b, W2b, b2b, Wfc, bfc):
    src = edge_index[0].astype(jnp.int32)
    dst = edge_index[1].astype(jnp.int32)
    packed = src | (dst << SHIFT)
    pad_i = jnp.arange(EP - E, dtype=jnp.int32) % (NP - N)
    pk3 = jnp.concatenate([packed, pad_i | ((N + pad_i) << SHIFT)]
                          ).reshape(NW, NCH, CH)
    zrows = jnp.zeros((NP, D), jnp.float32)
    b1a2, b2a2, b1b2, b2b2 = (b.reshape(1, D) for b in (b1a, b2a, b1b, b2b))
    Wfcp = jnp.pad(Wfc, ((0, 0), (0, D - Wfc.shape[1])))
    bfcp = jnp.pad(bfc, (0, D - bfc.shape[0])).reshape(1, D)

    p1 = _sc_scatter_add(x, pk3, zrows)
    h1 = _mlp(x, p1, W1a, b1a2, W2a, b2a2)
    p2 = _sc_scatter_add(h1, pk3, zrows)
    o = _mlp_fc(h1, p2, W1b, b1b2, W2b, b2b2, Wfcp, bfcp)
    return o[:, :3]


# speedup vs baseline: 1.0327x; 1.0327x over previous
import functools

import jax
import jax.numpy as jnp
from jax import lax
from jax.experimental import pallas as pl
from jax.experimental.pallas import tpu as pltpu
from jax.experimental.pallas import tpu_sc as plsc

N = 10000
D = 128
E = 320000
NC = 2
NS = 16
NW = NC * NS
CH = 128
NCH = 80
EP = NW * NCH * CH
NP = 10240
RPT = NP // NS
PAD_DST = NP - 1
SHIFT = 14


def _sc_scatter_add(h, pk3, zrows):
    mesh = plsc.VectorSubcoreMesh(
        core_axis_name="c", subcore_axis_name="s", num_cores=NC, num_subcores=NS
    )

    @functools.partial(
        pl.kernel,
        out_type=jax.ShapeDtypeStruct((NC, NP, D), jnp.float32),
        mesh=mesh,
        scratch_types=[
            pltpu.VMEM((NCH, CH), jnp.int32),
            pltpu.VMEM((2, CH), jnp.int32),
            pltpu.VMEM((2, CH), jnp.int32),
            pltpu.VMEM((2, CH, D), jnp.float32),
            pltpu.VMEM_SHARED((NP, D), jnp.float32),
            pltpu.SemaphoreType.DMA((2,)),
            pltpu.SemaphoreType.DMA((2,)),
        ],
    )
    def k(h_hbm, pk_hbm, z_hbm, out_hbm,
          pk_v, srcb_v, dstb_v, rows_v, acc_sh, gsem, ssem):
        c = lax.axis_index("c")
        s = lax.axis_index("s")
        wid = c * NS + s
        pltpu.sync_copy(pk_hbm.at[wid], pk_v)
        pltpu.sync_copy(z_hbm.at[pl.ds(s * RPT, RPT)],
                        acc_sh.at[pl.ds(s * RPT, RPT)])
        plsc.subcore_barrier()

        def unpack(j, slot):
            for kk in range(CH // 16):
                v = pk_v[j, pl.ds(kk * 16, 16)]
                srcb_v[slot, pl.ds(kk * 16, 16)] = v & ((1 << SHIFT) - 1)
                dstb_v[slot, pl.ds(kk * 16, 16)] = lax.shift_right_logical(
                    v, SHIFT)

        def gather_start(b):
            pltpu.async_copy(h_hbm.at[srcb_v.at[b]], rows_v.at[b], gsem.at[b])

        def gather_wait(b):
            pltpu.make_async_copy(
                h_hbm.at[srcb_v.at[b]], rows_v.at[b], gsem.at[b]).wait()

        def scatter_start(b):
            pltpu.async_copy(
                rows_v.at[b], acc_sh.at[dstb_v.at[b]], ssem.at[b], add=True)

        def scatter_wait(b):
            pltpu.make_async_copy(
                rows_v.at[b], acc_sh.at[dstb_v.at[b]], ssem.at[b]).wait()

        unpack(0, 0)
        gather_start(0)

        def body(j, carry):
            b = lax.rem(j, 2)
            nb = lax.rem(j + 1, 2)

            @pl.when(j >= 1)
            def _():
                scatter_wait(nb)

            @pl.when(j + 1 < NCH)
            def _():
                unpack(j + 1, nb)
                gather_start(nb)

            gather_wait(b)
            scatter_start(b)
            return carry

        lax.fori_loop(0, NCH, body, 0)
        scatter_wait(lax.rem(NCH - 1, 2))
        plsc.subcore_barrier()
        pltpu.sync_copy(acc_sh.at[pl.ds(s * RPT, RPT)],
                        out_hbm.at[c, pl.ds(s * RPT, RPT)])

    return k(h, pk3, zrows)


def _mlp(h, p, W1, b1, W2, b2):
    def body(h_ref, p_ref, w1_ref, b1_ref, w2_ref, b2_ref, o_ref):
        u = h_ref[...] + p_ref[0, :N] + p_ref[1, :N]
        t = jnp.dot(u, w1_ref[...], preferred_element_type=jnp.float32)
        t = jnp.maximum(t + b1_ref[...], 0.0)
        v = jnp.dot(t, w2_ref[...], preferred_element_type=jnp.float32)
        o_ref[...] = jnp.maximum(v + b2_ref[...], 0.0)

    return pl.pallas_call(
        body,
        out_shape=jax.ShapeDtypeStruct((N, D), jnp.float32),
    )(h, p, W1, b1, W2, b2)


def _mlp_fc(h, p, W1, b1, W2, b2, Wfcp, bfcp):
    def body(h_ref, p_ref, w1_ref, b1_ref, w2_ref, b2_ref,
             wfc_ref, bfc_ref, o_ref):
        u = h_ref[...] + p_ref[0, :N] + p_ref[1, :N]
        t = jnp.dot(u, w1_ref[...], preferred_element_type=jnp.float32)
        t = jnp.maximum(t + b1_ref[...], 0.0)
        v = jnp.dot(t, w2_ref[...], preferred_element_type=jnp.float32)
        h2 = jnp.maximum(v + b2_ref[...], 0.0)
        o = jnp.dot(h2, wfc_ref[...], preferred_element_type=jnp.float32)
        o_ref[...] = o + bfc_ref[...]

    return pl.pallas_call(
        body,
        out_shape=jax.ShapeDtypeStruct((N, D), jnp.float32),
    )(h, p, W1, b1, W2, b2, Wfcp, bfcp)


def kernel(x, edge_index, W1a, b1a, W2a, b2a, W1b, b1b, W2b, b2b, Wfc, bfc):
    src = edge_index[0].astype(jnp.int32)
    dst = edge_index[1].astype(jnp.int32)
    packed = src | (dst << SHIFT)
    pad_i = jnp.arange(EP - E, dtype=jnp.int32) % (NP - N)
    pk3 = jnp.concatenate([packed, pad_i | ((N + pad_i) << SHIFT)]
                          ).reshape(NW, NCH, CH)
    zrows = jnp.zeros((NP, D), jnp.float32)
    b1a2, b2a2, b1b2, b2b2 = (b.reshape(1, D) for b in (b1a, b2a, b1b, b2b))
    Wfcp = jnp.pad(Wfc, ((0, 0), (0, D - Wfc.shape[1])))
    bfcp = jnp.pad(bfc, (0, D - bfc.shape[0])).reshape(1, D)

    p1 = _sc_scatter_add(x, pk3, zrows)
    h1 = _mlp(x, p1, W1a, b1a2, W2a, b2a2)
    p2 = _sc_scatter_add(h1, pk3, zrows)
    o = _mlp_fc(h1, p2, W1b, b1b2, W2b, b2b2, Wfcp, bfcp)
    return o[:, :3]
